# Initial kernel scaffold; baseline (speedup 1.0000x reference)
import jax
import jax.numpy as jnp
from jax.experimental import pallas as pl


def kernel(emb, W_in, W_out):
    raise NotImplementedError("write your pallas kernel here")


# speedup vs baseline: 2.2680x; 2.2680x over previous
import numpy as np
import jax
import jax.numpy as jnp
from jax.experimental import pallas as pl
from jax.experimental.pallas import tpu as pltpu

ALPHA = 3.0
K = 20

_K0 = np.uint32(1832780943)
_K1 = np.uint32(270669613)
_KS2 = np.uint32(_K0 ^ _K1 ^ np.uint32(0x1BD11BDA))
_ROTS = ((13, 15, 26, 6), (17, 29, 16, 24))


def _rotl(x, r):
    return (x << np.uint32(r)) | (x >> np.uint32(32 - r))


def _threefry_bits(idx_u32):
    x0 = jnp.full(idx_u32.shape, _K0, jnp.uint32)
    x1 = idx_u32 + _K1
    ks = (_K1, _KS2, _K0)
    for i in range(5):
        for r in _ROTS[i % 2]:
            x0 = x0 + x1
            x1 = _rotl(x1, r) ^ x0
        x0 = x0 + ks[i % 3]
        x1 = x1 + ks[(i + 1) % 3] + np.uint32(i + 1)
    return x0 ^ x1


def _mlp_kernel(emb_ref, win_ref, wout_ref, m1_ref, m2_ref):
    e = emb_ref[...]
    dn = (((1,), (1,)), ((), ()))
    p = jax.lax.Precision.HIGHEST
    m1_ref[...] = jnp.tanh(
        jax.lax.dot_general(e, win_ref[...], dn, precision=p,
                            preferred_element_type=jnp.float32) * ALPHA)
    m2_ref[...] = jnp.tanh(
        jax.lax.dot_general(e, wout_ref[...], dn, precision=p,
                            preferred_element_type=jnp.float32) * ALPHA)


def _graph_kernel(m1r_ref, m2r_ref, m1_ref, m2_ref, out_ref, *, n, rows):
    dn = (((1,), (1,)), ((), ()))
    p = jax.lax.Precision.HIGHEST
    a = (jax.lax.dot_general(m1r_ref[...], m2_ref[...], dn, precision=p,
                             preferred_element_type=jnp.float32)
         - jax.lax.dot_general(m2r_ref[...], m1_ref[...], dn, precision=p,
                               preferred_element_type=jnp.float32))
    adj = jax.nn.relu(jnp.tanh(a * ALPHA))

    r0 = pl.program_id(0) * rows
    row = jax.lax.broadcasted_iota(jnp.int32, (rows, n), 0) + r0
    col = jax.lax.broadcasted_iota(jnp.int32, (rows, n), 1)
    idx = (row * n + col).astype(jnp.uint32)
    bits = _threefry_bits(idx)
    u = jax.lax.bitcast_convert_type(
        (bits >> np.uint32(9)) | np.uint32(0x3F800000), jnp.float32) - 1.0
    v = adj + u * np.float32(0.01)

    work = v
    sel = jnp.zeros((rows, n), jnp.bool_)
    big = jnp.int32(n)
    for _ in range(K):
        m = jnp.max(work, axis=1, keepdims=True)
        cand = jnp.where(work == m, col, big)
        cmin = jnp.min(cand, axis=1, keepdims=True)
        pick = cand == cmin
        sel = jnp.logical_or(sel, pick)
        work = jnp.where(pick, -jnp.inf, work)
    out_ref[...] = jnp.where(sel, adj, 0.0)


def _build(n, dim, rows):
    mlp = pl.pallas_call(
        _mlp_kernel,
        out_shape=(jax.ShapeDtypeStruct((n, dim), jnp.float32),
                   jax.ShapeDtypeStruct((n, dim), jnp.float32)),
    )

    import functools
    body = functools.partial(_graph_kernel, n=n, rows=rows)
    graph = pl.pallas_call(
        body,
        grid=(n // rows,),
        in_specs=[
            pl.BlockSpec((rows, dim), lambda i: (i, 0)),
            pl.BlockSpec((rows, dim), lambda i: (i, 0)),
            pl.BlockSpec((n, dim), lambda i: (0, 0)),
            pl.BlockSpec((n, dim), lambda i: (0, 0)),
        ],
        out_specs=pl.BlockSpec((rows, n), lambda i: (i, 0)),
        out_shape=jax.ShapeDtypeStruct((n, n), jnp.float32),
    )
    return mlp, graph


def kernel(emb, W_in, W_out):
    n, dim = emb.shape
    rows = 40 if n % 40 == 0 else 8
    mlp, graph = _build(n, dim, rows)
    m1, m2 = mlp(emb, W_in, W_out)
    adj = graph(m1, m2, m1, m2)
    return adj.reshape(1, 1, n, n)


# speedup vs baseline: 2.3839x; 1.0511x over previous
import functools

import numpy as np
import jax
import jax.numpy as jnp
from jax.experimental import pallas as pl

ALPHA = 3.0
K = 20

_K0 = np.uint32(1832780943)
_K1 = np.uint32(270669613)
_KS2 = np.uint32(_K0 ^ _K1 ^ np.uint32(0x1BD11BDA))
_ROTS = ((13, 15, 26, 6), (17, 29, 16, 24))


def _threefry_bits(idx_u32):
    x0 = jnp.full(idx_u32.shape, _K0, jnp.uint32)
    x1 = idx_u32 + _K1
    ks = (_K1, _KS2, _K0)
    for i in range(5):
        for r in _ROTS[i % 2]:
            x0 = x0 + x1
            x1 = ((x1 << np.uint32(r)) | (x1 >> np.uint32(32 - r))) ^ x0
        x0 = x0 + ks[i % 3]
        x1 = x1 + ks[(i + 1) % 3] + np.uint32(i + 1)
    return x0 ^ x1


def _mlp_kernel(embT_ref, win_ref, wout_ref, m1T_ref, m2T_ref, *, dim):
    embT = embT_ref[...]
    win = win_ref[...]
    wout = wout_ref[...]
    acc1 = jnp.zeros(m1T_ref.shape, jnp.float32)
    acc2 = jnp.zeros(m2T_ref.shape, jnp.float32)
    for k in range(dim):
        e_row = embT[k : k + 1, :]
        acc1 = acc1 + win[:, k : k + 1] * e_row
        acc2 = acc2 + wout[:, k : k + 1] * e_row
    m1T_ref[...] = jnp.tanh(ALPHA * acc1)
    m2T_ref[...] = jnp.tanh(ALPHA * acc2)


_DN = (((1,), (1,)), ((), ()))


def _adj_kernel(m1r_ref, m2r_ref, m1_ref, m2_ref, a_ref):
    A = jax.lax.dot_general(m1r_ref[...], m2_ref[...], _DN,
                            precision=jax.lax.Precision.DEFAULT,
                            preferred_element_type=jnp.float32)
    B = jax.lax.dot_general(m2r_ref[...], m1_ref[...], _DN,
                            precision=jax.lax.Precision.DEFAULT,
                            preferred_element_type=jnp.float32)
    a_ref[...] = A - B


def _select_kernel(adj_ref, out_ref, *, n, rows):
    adj = adj_ref[...]
    r0 = pl.program_id(0) * rows
    row = jax.lax.broadcasted_iota(jnp.int32, (rows, n), 0) + r0
    col = jax.lax.broadcasted_iota(jnp.int32, (rows, n), 1)
    idx = (row * n + col).astype(jnp.uint32)
    bits = _threefry_bits(idx)
    u = jax.lax.bitcast_convert_type(
        (bits >> np.uint32(9)) | np.uint32(0x3F800000), jnp.float32) - 1.0
    v = adj + u * np.float32(0.01)

    work = v
    sel = jnp.zeros((rows, n), jnp.bool_)
    big = jnp.int32(n)
    for _ in range(K):
        m = jnp.max(work, axis=1, keepdims=True)
        cand = jnp.where(work == m, col, big)
        cmin = jnp.min(cand, axis=1, keepdims=True)
        pick = cand == cmin
        sel = jnp.logical_or(sel, pick)
        work = jnp.where(pick, -jnp.inf, work)
    out_ref[...] = jnp.where(sel, adj, 0.0)


def _build(n, dim, rows_dot, rows_sel):
    mlp = pl.pallas_call(
        functools.partial(_mlp_kernel, dim=dim),
        out_shape=(jax.ShapeDtypeStruct((dim, n), jnp.float32),
                   jax.ShapeDtypeStruct((dim, n), jnp.float32)),
    )
    adj_dot = pl.pallas_call(
        _adj_kernel,
        grid=(n // rows_dot,),
        in_specs=[
            pl.BlockSpec((rows_dot, dim), lambda i: (i, 0)),
            pl.BlockSpec((rows_dot, dim), lambda i: (i, 0)),
            pl.BlockSpec((n, dim), lambda i: (0, 0)),
            pl.BlockSpec((n, dim), lambda i: (0, 0)),
        ],
        out_specs=pl.BlockSpec((rows_dot, n), lambda i: (i, 0)),
        out_shape=jax.ShapeDtypeStruct((n, n), jnp.float32),
    )
    select = pl.pallas_call(
        functools.partial(_select_kernel, n=n, rows=rows_sel),
        grid=(n // rows_sel,),
        in_specs=[pl.BlockSpec((rows_sel, n), lambda i: (i, 0))],
        out_specs=pl.BlockSpec((rows_sel, n), lambda i: (i, 0)),
        out_shape=jax.ShapeDtypeStruct((n, n), jnp.float32),
    )
    return mlp, adj_dot, select


def kernel(emb, W_in, W_out):
    n, dim = emb.shape
    rows_dot = 200 if n % 200 == 0 else 8
    rows_sel = 40 if n % 40 == 0 else 8
    _, adj_dot, select = _build(n, dim, rows_dot, rows_sel)
    m1 = jnp.tanh(ALPHA * (emb @ W_in.T))
    m2 = jnp.tanh(ALPHA * (emb @ W_out.T))
    a = adj_dot(m1, m2, m1, m2)
    adj = jax.nn.relu(jnp.tanh(ALPHA * a))
    out = select(adj)
    return out.reshape(1, 1, n, n)


# speedup vs baseline: 2.3843x; 1.0002x over previous
import functools

import numpy as np
import jax
import jax.numpy as jnp
from jax.experimental import pallas as pl

ALPHA = 3.0
K = 20

_K0 = np.uint32(1832780943)
_K1 = np.uint32(270669613)
_KS2 = np.uint32(_K0 ^ _K1 ^ np.uint32(0x1BD11BDA))
_ROTS = ((13, 15, 26, 6), (17, 29, 16, 24))


def _threefry_bits(idx_u32):
    x0 = jnp.full(idx_u32.shape, _K0, jnp.uint32)
    x1 = idx_u32 + _K1
    ks = (_K1, _KS2, _K0)
    for i in range(5):
        for r in _ROTS[i % 2]:
            x0 = x0 + x1
            x1 = ((x1 << np.uint32(r)) | (x1 >> np.uint32(32 - r))) ^ x0
        x0 = x0 + ks[i % 3]
        x1 = x1 + ks[(i + 1) % 3] + np.uint32(i + 1)
    return x0 ^ x1


_DN = (((1,), (1,)), ((), ()))


def _adj_kernel(m1r_ref, m2r_ref, m1_ref, m2_ref, a_ref):
    A = jax.lax.dot_general(m1r_ref[...], m2_ref[...], _DN,
                            precision=jax.lax.Precision.DEFAULT,
                            preferred_element_type=jnp.float32)
    B = jax.lax.dot_general(m2r_ref[...], m1_ref[...], _DN,
                            precision=jax.lax.Precision.DEFAULT,
                            preferred_element_type=jnp.float32)
    a_ref[...] = A - B


def _select_kernel(adj_ref, out_ref, *, n, rows):
    adj = adj_ref[...]
    r0 = pl.program_id(0) * rows
    row = jax.lax.broadcasted_iota(jnp.int32, (rows, n), 0) + r0
    col = jax.lax.broadcasted_iota(jnp.int32, (rows, n), 1)
    idx = (row * n + col).astype(jnp.uint32)
    bits = _threefry_bits(idx)
    u = jax.lax.bitcast_convert_type(
        (bits >> np.uint32(9)) | np.uint32(0x3F800000), jnp.float32) - 1.0
    v = adj + u * np.float32(0.01)

    work = v
    sel = jnp.zeros((rows, n), jnp.bool_)
    big = jnp.int32(n)
    for _ in range(K):
        m = jnp.max(work, axis=1, keepdims=True)
        cand = jnp.where(work == m, col, big)
        cmin = jnp.min(cand, axis=1, keepdims=True)
        pick = cand == cmin
        sel = jnp.logical_or(sel, pick)
        work = jnp.where(pick, -jnp.inf, work)
    out_ref[...] = jnp.where(sel, adj, 0.0)


def _build(n, dim, rows_dot, rows_sel):
    adj_dot = pl.pallas_call(
        _adj_kernel,
        grid=(n // rows_dot,),
        in_specs=[
            pl.BlockSpec((rows_dot, dim), lambda i: (i, 0)),
            pl.BlockSpec((rows_dot, dim), lambda i: (i, 0)),
            pl.BlockSpec((n, dim), lambda i: (0, 0)),
            pl.BlockSpec((n, dim), lambda i: (0, 0)),
        ],
        out_specs=pl.BlockSpec((rows_dot, n), lambda i: (i, 0)),
        out_shape=jax.ShapeDtypeStruct((n, n), jnp.float32),
    )
    select = pl.pallas_call(
        functools.partial(_select_kernel, n=n, rows=rows_sel),
        grid=(n // rows_sel,),
        in_specs=[pl.BlockSpec((rows_sel, n), lambda i: (i, 0))],
        out_specs=pl.BlockSpec((rows_sel, n), lambda i: (i, 0)),
        out_shape=jax.ShapeDtypeStruct((n, n), jnp.float32),
    )
    return adj_dot, select


def kernel(emb, W_in, W_out):
    n, dim = emb.shape
    rows_dot = 200 if n % 200 == 0 else 8
    rows_sel = 40 if n % 40 == 0 else 8
    adj_dot, select = _build(n, dim, rows_dot, rows_sel)
    m1 = jnp.tanh(ALPHA * (emb @ W_in.T))
    m2 = jnp.tanh(ALPHA * (emb @ W_out.T))
    a = adj_dot(m1, m2, m1, m2)
    adj = jax.nn.relu(jnp.tanh(ALPHA * a))
    out = select(adj)
    return out.reshape(1, 1, n, n)
